# baseline (speedup 1.0000x reference)
import functools

import jax
import jax.numpy as jnp
from jax import lax
from jax.experimental import pallas as pl
from jax.experimental.pallas import tpu as pltpu
from jax.experimental.pallas import tpu_sc as plsc

B = 4
S = 4096
D = 768
LANES = 16
NUM_CORES = 2
NUM_SUBCORES = 16
NW = NUM_CORES * NUM_SUBCORES
SPW = S // NW
C = 64
VECS_PER_ROW = D // LANES


def _body(x_hbm, table_hbm, pos_hbm, out_hbm, idx_v, pos_v, rows_v, sem):
    wid = lax.axis_index("s") * NUM_CORES + lax.axis_index("c")
    s0 = wid * SPW
    for sc_chunk in range(SPW // C):
        s_base = s0 + sc_chunk * C
        pltpu.sync_copy(pos_hbm.at[pl.ds(s_base, C)], pos_v)
        for b in range(B):
            base = b * S + s_base
            pltpu.sync_copy(x_hbm.at[pl.ds(base, C)], idx_v)
            pltpu.async_copy(table_hbm.at[idx_v], rows_v, sem).wait()

            def add_row(r, carry):
                for j in range(VECS_PER_ROW):
                    sl = pl.ds(j * LANES, LANES)
                    rows_v[r, sl] = rows_v[r, sl] + pos_v[r, sl]
                return carry

            lax.fori_loop(0, C, add_row, 0)
            pltpu.sync_copy(rows_v, out_hbm.at[pl.ds(base, C)])


@jax.jit
def _embed(x_flat, table, pos_table):
    mesh = plsc.VectorSubcoreMesh(core_axis_name="c", subcore_axis_name="s")
    kfn = functools.partial(
        pl.kernel,
        out_type=jax.ShapeDtypeStruct((B * S, D), jnp.float32),
        mesh=mesh,
        scratch_types=[
            pltpu.VMEM((C,), jnp.int32),
            pltpu.VMEM((C, D), jnp.float32),
            pltpu.VMEM((C, D), jnp.float32),
            pltpu.SemaphoreType.DMA,
        ],
    )(_body)
    return kfn(x_flat, table, pos_table)


def kernel(x, table, pos_table):
    x_flat = x.reshape(B * S).astype(jnp.int32)
    out = _embed(x_flat, table, pos_table)
    return out.reshape(B, S, D)
